# baseline (speedup 1.0000x reference)
import functools

import jax
import jax.numpy as jnp
from jax import lax
from jax.experimental import pallas as pl
from jax.experimental.pallas import tpu as pltpu
from jax.experimental.pallas import tpu_sc as plsc

_NUM_K = 1024
_DIM = 64
_TILE = 1024
_COMMITMENT_COST = 0.25
_GATHER_CHUNK = 128
_PAD_DIM = 128


def _vq_idx_body(x_ref, xsq_ref, w_ref, idx_ref, loss_ref, w2n_ref, wsq_ref):
    @pl.when(pl.program_id(0) == 0)
    def _init():
        w = w_ref[...]
        w2n_ref[...] = -2.0 * w
        wsq_ref[...] = jnp.sum(w * w, axis=1, keepdims=True)
        loss_ref[...] = jnp.zeros_like(loss_ref)

    x = x_ref[0]
    xsq = xsq_ref[0]
    mm2 = jax.lax.dot_general(
        w2n_ref[...], x, (((1,), (0,)), ((), ())),
        preferred_element_type=jnp.float32)
    wsq = wsq_ref[...]
    minval = jnp.min((wsq + xsq) + mm2,
                     axis=0, keepdims=True)
    kiota = jax.lax.broadcasted_iota(
        jnp.int32, (_NUM_K, _TILE), 0).astype(jnp.float32)
    idxf = jnp.min(jnp.where(((xsq + wsq) + mm2) == minval,
                             kiota, float(_NUM_K)),
                   axis=0, keepdims=True)
    idx_ref[0] = idxf.astype(jnp.int32)
    loss_ref[...] = loss_ref[...] + jnp.sum(minval)


def _argmin_and_loss(x, xsq, weight):
    b = x.shape[0]
    hw = x.shape[2]
    return pl.pallas_call(
        _vq_idx_body,
        grid=(b,),
        in_specs=[
            pl.BlockSpec((1, _DIM, _TILE), lambda i: (i, 0, 0)),
            pl.BlockSpec((1, 1, _TILE), lambda i: (i, 0, 0)),
            pl.BlockSpec((_NUM_K, _DIM), lambda i: (0, 0)),
        ],
        out_specs=[
            pl.BlockSpec((1, 1, hw), lambda i: (i, 0, 0)),
            pl.BlockSpec((1, 1), lambda i: (0, 0)),
        ],
        out_shape=[
            jax.ShapeDtypeStruct((b, 1, hw), jnp.int32),
            jax.ShapeDtypeStruct((1, 1), jnp.float32),
        ],
        scratch_shapes=[
            pltpu.VMEM((_NUM_K, _DIM), jnp.float32),
            pltpu.VMEM((_NUM_K, 1), jnp.float32),
        ],
    )(x, xsq, weight)


def _make_sc_gather(n_rows):
    info = plsc.get_sparse_core_info()
    num_workers = info.num_cores * info.num_subcores
    rows_per_worker = n_rows // num_workers
    n_chunks = rows_per_worker // _GATHER_CHUNK
    mesh = plsc.VectorSubcoreMesh(core_axis_name="c", subcore_axis_name="s")

    @functools.partial(
        pl.kernel, mesh=mesh,
        out_type=jax.ShapeDtypeStruct((n_rows, _DIM), jnp.float32),
        compiler_params=pltpu.CompilerParams(use_tc_tiling_on_sc=False),
        scratch_types=[
            pltpu.VMEM((n_chunks, _GATHER_CHUNK), jnp.int32),
            pltpu.VMEM((rows_per_worker, _DIM), jnp.float32),
            pltpu.SemaphoreType.DMA,
            pltpu.SemaphoreType.DMA,
        ],
    )
    def _gather(idx_hbm, table_hbm, out_hbm, idx_v, rows_v, gsem, ssem):
        wid = lax.axis_index("s") * info.num_cores + lax.axis_index("c")
        base = wid * rows_per_worker
        pltpu.sync_copy(idx_hbm.at[pl.ds(wid * n_chunks, n_chunks)], idx_v)
        gathers = [
            pltpu.async_copy(
                table_hbm.at[idx_v.at[j]],
                rows_v.at[pl.ds(j * _GATHER_CHUNK, _GATHER_CHUNK)],
                gsem)
            for j in range(n_chunks)
        ]
        for cp in gathers:
            cp.wait()
        pltpu.async_copy(rows_v, out_hbm.at[pl.ds(base, rows_per_worker)],
                         ssem).wait()

    return _gather


def kernel(inputs, weight):
    b, c, h, w_sz = inputs.shape
    hw = h * w_sz
    n = b * hw

    x = inputs.reshape(b, c, hw)
    xsq = jnp.sum(x * x, axis=1, keepdims=True)
    idx, loss_acc = _argmin_and_loss(x, xsq, weight)
    idx2d = idx.reshape(n // _GATHER_CHUNK, _GATHER_CHUNK)
    q_flat = _make_sc_gather(n)(idx2d, weight)

    n_total = n * _DIM
    mean_sq = loss_acc[0, 0] / n_total
    loss = mean_sq + _COMMITMENT_COST * mean_sq
    quantized_out = jnp.transpose(
        q_flat.reshape(b, h, w_sz, c), (0, 3, 1, 2))
    return (quantized_out, loss)


# speedup vs baseline: 1.0435x; 1.0435x over previous
import functools

import jax
import jax.numpy as jnp
from jax import lax
from jax.experimental import pallas as pl
from jax.experimental.pallas import tpu as pltpu
from jax.experimental.pallas import tpu_sc as plsc

_NUM_K = 1024
_DIM = 64
_TILE = 1024
_COMMITMENT_COST = 0.25
_GATHER_CHUNK = 128
_PAD_DIM = 128


def _vq_idx_body(x_ref, xsq_ref, w_ref, idx_ref, loss_ref, w2n_ref, wsq_ref):
    @pl.when(pl.program_id(0) == 0)
    def _init():
        w = w_ref[...]
        w2n_ref[...] = -2.0 * w
        wsq_ref[...] = jnp.sum(w * w, axis=1, keepdims=True)
        loss_ref[...] = jnp.zeros_like(loss_ref)

    x = x_ref[0]
    xsq = xsq_ref[0]
    mm2 = jax.lax.dot_general(
        w2n_ref[...], x, (((1,), (0,)), ((), ())),
        preferred_element_type=jnp.float32)
    wsq = wsq_ref[...]
    minval = jnp.min((wsq + xsq) + mm2,
                     axis=0, keepdims=True)
    kiota = jax.lax.broadcasted_iota(
        jnp.int32, (_NUM_K, _TILE), 0).astype(jnp.float32)
    idxf = jnp.min(jnp.where(((xsq + wsq) + mm2) == minval,
                             kiota, float(_NUM_K)),
                   axis=0, keepdims=True)
    idx_ref[0] = idxf.astype(jnp.int32)
    loss_ref[...] = loss_ref[...] + jnp.sum(minval)


def _argmin_and_loss(x, xsq, weight):
    b = x.shape[0]
    hw = x.shape[2]
    return pl.pallas_call(
        _vq_idx_body,
        grid=(b,),
        in_specs=[
            pl.BlockSpec((1, _DIM, _TILE), lambda i: (i, 0, 0)),
            pl.BlockSpec((1, 1, _TILE), lambda i: (i, 0, 0)),
            pl.BlockSpec((_NUM_K, _DIM), lambda i: (0, 0)),
        ],
        out_specs=[
            pl.BlockSpec((1, 1, hw), lambda i: (i, 0, 0)),
            pl.BlockSpec((1, 1), lambda i: (0, 0)),
        ],
        out_shape=[
            jax.ShapeDtypeStruct((b, 1, hw), jnp.int32),
            jax.ShapeDtypeStruct((1, 1), jnp.float32),
        ],
        scratch_shapes=[
            pltpu.VMEM((_NUM_K, _DIM), jnp.float32),
            pltpu.VMEM((_NUM_K, 1), jnp.float32),
        ],
        compiler_params=pltpu.CompilerParams(allow_input_fusion=[0, 1]),
    )(x, xsq, weight)


def _make_sc_gather(n_rows):
    info = plsc.get_sparse_core_info()
    num_workers = info.num_cores * info.num_subcores
    rows_per_worker = n_rows // num_workers
    n_chunks = rows_per_worker // _GATHER_CHUNK
    mesh = plsc.VectorSubcoreMesh(core_axis_name="c", subcore_axis_name="s")

    @functools.partial(
        pl.kernel, mesh=mesh,
        out_type=jax.ShapeDtypeStruct((n_rows, _PAD_DIM), jnp.float32),
        scratch_types=[
            pltpu.VMEM((n_chunks, _GATHER_CHUNK), jnp.int32),
            pltpu.VMEM((rows_per_worker, _PAD_DIM), jnp.float32),
            pltpu.SemaphoreType.DMA,
            pltpu.SemaphoreType.DMA,
        ],
    )
    def _gather(idx_hbm, table_hbm, out_hbm, idx_v, rows_v, gsem, ssem):
        wid = lax.axis_index("s") * info.num_cores + lax.axis_index("c")
        base = wid * rows_per_worker
        pltpu.sync_copy(idx_hbm.at[pl.ds(wid * n_chunks, n_chunks)], idx_v)
        gathers = [
            pltpu.async_copy(
                table_hbm.at[idx_v.at[j]],
                rows_v.at[pl.ds(j * _GATHER_CHUNK, _GATHER_CHUNK)],
                gsem)
            for j in range(n_chunks)
        ]
        for cp in gathers:
            cp.wait()
        pltpu.async_copy(rows_v, out_hbm.at[pl.ds(base, rows_per_worker)],
                         ssem).wait()

    return _gather


def kernel(inputs, weight):
    b, c, h, w_sz = inputs.shape
    hw = h * w_sz
    n = b * hw

    x = inputs.reshape(b, c, hw)
    xsq = jnp.sum(x * x, axis=1, keepdims=True)
    idx, loss_acc = _argmin_and_loss(x, xsq, weight)
    wpad = jnp.pad(weight, ((0, 0), (0, _PAD_DIM - _DIM)))
    idx2d = idx.reshape(n // _GATHER_CHUNK, _GATHER_CHUNK)
    q_pad = _make_sc_gather(n)(idx2d, wpad)
    q_flat = q_pad[:, :_DIM]

    n_total = n * _DIM
    mean_sq = loss_acc[0, 0] / n_total
    loss = mean_sq + _COMMITMENT_COST * mean_sq
    quantized_out = jnp.transpose(
        q_flat.reshape(b, h, w_sz, c), (0, 3, 1, 2))
    return (quantized_out, loss)
